# baseline (speedup 1.0000x reference)
import functools

import jax
import jax.numpy as jnp
from jax.experimental import pallas as pl
from jax.experimental.pallas import tpu as pltpu


def _body(lp_ref, lt_ref, ct_ref, sum_ref, cnt_ref, acc_ref, pacc_ref):
    c = pl.program_id(0)
    nsteps = pl.num_programs(0)

    @pl.when(c == 0)
    def _init():
        acc_ref[...] = jnp.zeros_like(acc_ref)
        pacc_ref[...] = jnp.zeros_like(pacc_ref)

    d = lp_ref[...] - lt_ref[...]
    ad = jnp.abs(d)
    sl1 = jnp.where(ad < 1.0, 0.5 * d * d, ad - 0.5)
    pos = (ct_ref[...] > 0).astype(jnp.float32)
    acc_ref[...] += sl1 * pos[:, None, :]
    pacc_ref[...] += pos

    @pl.when(c == nsteps - 1)
    def _finish():
        sum_ref[0, 0] = jnp.sum(acc_ref[...])
        cnt_ref[0, 0] = jnp.sum(pacc_ref[...])


@functools.partial(jax.jit, static_argnames=("interpret",))
def _ohem(loc_preds, loc_targets, cls_targets, interpret=False):
    B, A, L = loc_preds.shape
    lpT = jnp.transpose(loc_preds, (0, 2, 1))
    ltT = jnp.transpose(loc_targets, (0, 2, 1))
    CH = 4096
    grid = (A // CH,)
    ct = cls_targets.astype(jnp.int32)
    s, n = pl.pallas_call(
        _body,
        grid=grid,
        in_specs=[
            pl.BlockSpec((B, L, CH), lambda c: (0, 0, c)),
            pl.BlockSpec((B, L, CH), lambda c: (0, 0, c)),
            pl.BlockSpec((B, CH), lambda c: (0, c)),
        ],
        out_specs=[
            pl.BlockSpec(memory_space=pltpu.SMEM),
            pl.BlockSpec(memory_space=pltpu.SMEM),
        ],
        out_shape=[
            jax.ShapeDtypeStruct((1, 1), jnp.float32),
            jax.ShapeDtypeStruct((1, 1), jnp.float32),
        ],
        scratch_shapes=[
            pltpu.VMEM((B, L, CH), jnp.float32),
            pltpu.VMEM((B, CH), jnp.float32),
        ],
        interpret=interpret,
    )(lpT, ltT, ct)
    return 0.2 * s[0, 0] / n[0, 0]


def _sc_ohem(loc_preds, loc_targets, cls_targets):
    from jax import lax
    from jax.experimental.pallas import tpu_sc as plsc

    B, A, L = loc_preds.shape
    lpT = jnp.transpose(loc_preds, (0, 2, 1))
    ltT = jnp.transpose(loc_targets, (0, 2, 1))
    ct = cls_targets.astype(jnp.int32)
    NC, NL = 2, 16
    NW = 32
    CHA = 2048
    NCH = A // CHA

    @functools.partial(
        pl.kernel,
        mesh=plsc.VectorSubcoreMesh(core_axis_name="c", subcore_axis_name="s"),
        out_type=[
            jax.ShapeDtypeStruct((NW * NL,), jnp.float32),
            jax.ShapeDtypeStruct((NW * NL,), jnp.float32),
        ],
        scratch_types=[
            pltpu.VMEM((L, CHA), jnp.float32),
            pltpu.VMEM((L, CHA), jnp.float32),
            pltpu.VMEM((CHA,), jnp.int32),
            pltpu.VMEM((NL,), jnp.float32),
            pltpu.VMEM((NL,), jnp.float32),
        ],
    )
    def _sc_body(lp_hbm, lt_hbm, ct_hbm, sum_hbm, cnt_hbm,
                 lp_v, lt_v, ct_v, s_v, c_v):
        w = lax.axis_index("s") * NC + lax.axis_index("c")
        zero = jnp.zeros((NL,), jnp.float32)
        one = jnp.ones((NL,), jnp.float32)
        half = jnp.full((NL,), 0.5, jnp.float32)

        def chunk_body(ci, carry):
            acc, cnt = carry
            pltpu.sync_copy(lp_hbm.at[w, :, pl.ds(ci * CHA, CHA)], lp_v)
            pltpu.sync_copy(lt_hbm.at[w, :, pl.ds(ci * CHA, CHA)], lt_v)
            pltpu.sync_copy(ct_hbm.at[w, pl.ds(ci * CHA, CHA)], ct_v)

            def g_body(g, carry2):
                acc2, cnt2 = carry2
                m = ct_v[pl.ds(g * NL, NL)] > 0
                cnt2 = cnt2 + jnp.where(m, one, zero)
                for j in range(L):
                    d = lp_v[j, pl.ds(g * NL, NL)] - lt_v[j, pl.ds(g * NL, NL)]
                    ad = jnp.abs(d)
                    sl1 = jnp.where(ad < one, half * d * d, ad - half)
                    acc2 = acc2 + jnp.where(m, sl1, zero)
                return acc2, cnt2

            return lax.fori_loop(0, CHA // NL, g_body, (acc, cnt))

        acc, cnt = lax.fori_loop(0, NCH, chunk_body, (zero, zero))
        s_v[...] = acc
        c_v[...] = cnt
        pltpu.sync_copy(s_v, sum_hbm.at[pl.ds(w * NL, NL)])
        pltpu.sync_copy(c_v, cnt_hbm.at[pl.ds(w * NL, NL)])

    s, n = _sc_body(lpT, ltT, ct)
    return 0.2 * jnp.sum(s) / jnp.sum(n)


def kernel(loc_preds, loc_targets, cls_preds, cls_targets):
    return _sc_ohem(loc_preds, loc_targets, cls_targets)


# speedup vs baseline: 2.7364x; 2.7364x over previous
import functools

import jax
import jax.numpy as jnp
from jax import lax
from jax.experimental import pallas as pl
from jax.experimental.pallas import tpu as pltpu
from jax.experimental.pallas import tpu_sc as plsc

_CH_TC = 4096
_CHA_SC = 2048
_A_SC = 12288


def _tc_body(lp_ref, lt_ref, ct_ref, sum_ref, cnt_ref, acc_ref, pacc_ref):
    c = pl.program_id(0)
    nsteps = pl.num_programs(0)

    @pl.when(c == 0)
    def _init():
        acc_ref[...] = jnp.zeros_like(acc_ref)
        pacc_ref[...] = jnp.zeros_like(pacc_ref)

    d = lp_ref[...] - lt_ref[...]
    ad = jnp.abs(d)
    sl1 = jnp.where(ad < 1.0, 0.5 * d * d, ad - 0.5)
    pos = (ct_ref[...] > 0).astype(jnp.float32)
    acc_ref[...] += sl1 * pos[:, None, :]
    pacc_ref[...] += pos

    @pl.when(c == nsteps - 1)
    def _finish():
        sum_ref[0, 0] = jnp.sum(acc_ref[...])
        cnt_ref[0, 0] = jnp.sum(pacc_ref[...])


def _tc_part(lpT, ltT, ct, a_lo, a_hi):
    B, L, A = lpT.shape
    CH = _CH_TC
    nblk = (a_hi - a_lo) // CH
    blk0 = a_lo // CH
    s, n = pl.pallas_call(
        _tc_body,
        grid=(nblk,),
        in_specs=[
            pl.BlockSpec((B, L, CH), lambda c: (0, 0, c + blk0)),
            pl.BlockSpec((B, L, CH), lambda c: (0, 0, c + blk0)),
            pl.BlockSpec((B, CH), lambda c: (0, c + blk0)),
        ],
        out_specs=[
            pl.BlockSpec(memory_space=pltpu.SMEM),
            pl.BlockSpec(memory_space=pltpu.SMEM),
        ],
        out_shape=[
            jax.ShapeDtypeStruct((1, 1), jnp.float32),
            jax.ShapeDtypeStruct((1, 1), jnp.float32),
        ],
        scratch_shapes=[
            pltpu.VMEM((B, L, CH), jnp.float32),
            pltpu.VMEM((B, CH), jnp.float32),
        ],
    )(lpT, ltT, ct)
    return s[0, 0], n[0, 0]


def _sc_part(lpT, ltT, ct, a_lo, a_hi):
    B, L, A = lpT.shape
    NC, NL = 2, 16
    NW = 32
    CHA = _CHA_SC
    nch = (a_hi - a_lo) // CHA

    @functools.partial(
        pl.kernel,
        mesh=plsc.VectorSubcoreMesh(core_axis_name="c", subcore_axis_name="s"),
        out_type=[
            jax.ShapeDtypeStruct((NW * NL,), jnp.float32),
            jax.ShapeDtypeStruct((NW * NL,), jnp.float32),
        ],
        scratch_types=[
            pltpu.VMEM((L, CHA), jnp.float32),
            pltpu.VMEM((L, CHA), jnp.float32),
            pltpu.VMEM((CHA,), jnp.int32),
            pltpu.VMEM((NL,), jnp.float32),
            pltpu.VMEM((NL,), jnp.float32),
        ],
    )
    def _sc_body(lp_hbm, lt_hbm, ct_hbm, sum_hbm, cnt_hbm,
                 lp_v, lt_v, ct_v, s_v, c_v):
        w = lax.axis_index("s") * NC + lax.axis_index("c")
        zero = jnp.zeros((NL,), jnp.float32)
        one = jnp.ones((NL,), jnp.float32)
        half = jnp.full((NL,), 0.5, jnp.float32)

        def chunk_body(ci, carry):
            acc, cnt = carry
            a0 = a_lo + ci * CHA
            pltpu.sync_copy(lp_hbm.at[w, :, pl.ds(a0, CHA)], lp_v)
            pltpu.sync_copy(lt_hbm.at[w, :, pl.ds(a0, CHA)], lt_v)
            pltpu.sync_copy(ct_hbm.at[w, pl.ds(a0, CHA)], ct_v)

            def g_body(g, carry2):
                acc2, cnt2 = carry2
                m = ct_v[pl.ds(g * NL, NL)] > 0
                cnt2 = cnt2 + jnp.where(m, one, zero)
                for j in range(L):
                    d = lp_v[j, pl.ds(g * NL, NL)] - lt_v[j, pl.ds(g * NL, NL)]
                    ad = jnp.abs(d)
                    sl1 = jnp.where(ad < one, half * d * d, ad - half)
                    acc2 = acc2 + jnp.where(m, sl1, zero)
                return acc2, cnt2

            return lax.fori_loop(0, CHA // NL, g_body, (acc, cnt))

        acc, cnt = lax.fori_loop(0, nch, chunk_body, (zero, zero))
        s_v[...] = acc
        c_v[...] = cnt
        pltpu.sync_copy(s_v, sum_hbm.at[pl.ds(w * NL, NL)])
        pltpu.sync_copy(c_v, cnt_hbm.at[pl.ds(w * NL, NL)])

    s, n = _sc_body(lpT, ltT, ct)
    return jnp.sum(s), jnp.sum(n)


@jax.jit
def _ohem(loc_preds, loc_targets, cls_targets):
    B, A, L = loc_preds.shape
    lpT = jnp.transpose(loc_preds, (0, 2, 1))
    ltT = jnp.transpose(loc_targets, (0, 2, 1))
    ct = cls_targets.astype(jnp.int32)
    a_split = A - _A_SC
    s_sc, n_sc = _sc_part(lpT, ltT, ct, a_split, A)
    s_tc, n_tc = _tc_part(lpT, ltT, ct, 0, a_split)
    return 0.2 * (s_tc + s_sc) / (n_tc + n_sc)


def kernel(loc_preds, loc_targets, cls_preds, cls_targets):
    return _ohem(loc_preds, loc_targets, cls_targets)


# speedup vs baseline: 3.7213x; 1.3599x over previous
import functools

import jax
import jax.numpy as jnp
from jax.experimental import pallas as pl
from jax.experimental.pallas import tpu as pltpu


def _body(lp_ref, lt_ref, ct_ref, sum_ref, cnt_ref, acc_ref, pacc_ref):
    c = pl.program_id(0)
    nsteps = pl.num_programs(0)

    @pl.when(c == 0)
    def _init():
        acc_ref[...] = jnp.zeros_like(acc_ref)
        pacc_ref[...] = jnp.zeros_like(pacc_ref)

    d = lp_ref[...] - lt_ref[...]
    ad = jnp.abs(d)
    sl1 = jnp.where(ad < 1.0, 0.5 * d * d, ad - 0.5)
    pos = (ct_ref[...] > 0).astype(jnp.float32)
    acc_ref[...] += sl1 * pos[:, None, :]
    pacc_ref[...] += pos

    @pl.when(c == nsteps - 1)
    def _finish():
        sum_ref[0, 0] = jnp.sum(acc_ref[...])
        cnt_ref[0, 0] = jnp.sum(pacc_ref[...])


@functools.partial(jax.jit, static_argnames=("interpret",))
def _ohem(loc_preds, loc_targets, cls_targets, interpret=False):
    B, A, L = loc_preds.shape
    lpT = jnp.transpose(loc_preds, (0, 2, 1))
    ltT = jnp.transpose(loc_targets, (0, 2, 1))
    CH = 4096
    grid = (A // CH,)
    ct = cls_targets.astype(jnp.int32)
    s, n = pl.pallas_call(
        _body,
        grid=grid,
        in_specs=[
            pl.BlockSpec((B, L, CH), lambda c: (0, 0, c)),
            pl.BlockSpec((B, L, CH), lambda c: (0, 0, c)),
            pl.BlockSpec((B, CH), lambda c: (0, c)),
        ],
        out_specs=[
            pl.BlockSpec(memory_space=pltpu.SMEM),
            pl.BlockSpec(memory_space=pltpu.SMEM),
        ],
        out_shape=[
            jax.ShapeDtypeStruct((1, 1), jnp.float32),
            jax.ShapeDtypeStruct((1, 1), jnp.float32),
        ],
        scratch_shapes=[
            pltpu.VMEM((B, L, CH), jnp.float32),
            pltpu.VMEM((B, CH), jnp.float32),
        ],
        interpret=interpret,
    )(lpT, ltT, ct)
    return 0.2 * s[0, 0] / n[0, 0]


def kernel(loc_preds, loc_targets, cls_preds, cls_targets):
    return _ohem(loc_preds, loc_targets, cls_targets)
